# Initial kernel scaffold; baseline (speedup 1.0000x reference)
import jax
import jax.numpy as jnp
from jax.experimental import pallas as pl


def kernel(x, Wr, br, W1, b1, W2, b2):
    raise NotImplementedError("write your pallas kernel here")


# speedup vs baseline: 2.0731x; 2.0731x over previous
import jax
import jax.numpy as jnp
from jax import lax
from jax.experimental import pallas as pl
from jax.experimental.pallas import tpu as pltpu

DIM_ = 1024
NE_ = 8
HID_ = 512
NTOK_ = 4096
EPAD_ = 128
RTB_ = 1024


def _router_body(x_ref, wr_ref, br_ref, wfull_ref, aux_ref, psum_ref):
    tb = pl.program_id(0)
    xf = x_ref[...]
    lg = jnp.dot(xf, wr_ref[...], precision=lax.Precision.DEFAULT,
                 preferred_element_type=jnp.float32)
    lg = (lg + br_ref[...]) * 10.0
    col = lax.broadcasted_iota(jnp.int32, (RTB_, EPAD_), 1)
    valid = col < NE_
    lg = jnp.where(valid, lg, -1e30)
    m1 = jnp.max(lg, axis=1, keepdims=True)
    a1 = jnp.min(jnp.where(lg == m1, col, EPAD_), axis=1, keepdims=True)
    lg2 = jnp.where(col == a1, -1e30, lg)
    m2 = jnp.max(lg2, axis=1, keepdims=True)
    a2 = jnp.min(jnp.where(lg2 == m2, col, EPAD_), axis=1, keepdims=True)
    w1w = 1.0 / (1.0 + jnp.exp(m2 - m1))
    wfull_ref[...] = (jnp.where(col == a1, w1w, 0.0)
                      + jnp.where(col == a2, 1.0 - w1w, 0.0))
    ex = jnp.where(valid, jnp.exp(lg - m1), 0.0)
    p = ex / jnp.sum(ex, axis=1, keepdims=True)
    ps = jnp.sum(p, axis=0, keepdims=True)

    @pl.when(tb == 0)
    def _init():
        psum_ref[...] = jnp.zeros_like(psum_ref)

    psum_ref[...] += ps

    @pl.when(tb == pl.num_programs(0) - 1)
    def _fin():
        s = psum_ref[...]
        aux_ref[...] = (jnp.sum(s * s) / NE_ * 1e-5) * jnp.ones(
            (1, 1), jnp.float32)


def _expert_body(xbf_ref, wfull_ref, w1_ref, b1_ref, w2_ref, b2_ref,
                 out_ref):
    e = pl.program_id(0)

    @pl.when(e == 0)
    def _init():
        out_ref[...] = jnp.zeros_like(out_ref)

    col = lax.broadcasted_iota(jnp.int32, (NTOK_, EPAD_), 1)
    wcol = jnp.sum(jnp.where(col == e, wfull_ref[...], 0.0),
                   axis=1, keepdims=True)
    w1m = w1_ref[0]
    w2m = w2_ref[0]
    b1v = b1_ref[0]
    b2v = b2_ref[0]
    chunk = 1024
    for i in range(NTOK_ // chunk):
        rows = pl.ds(i * chunk, chunk)
        xc = xbf_ref[rows, :]
        hc = jnp.dot(xc, w1m, preferred_element_type=jnp.float32) + b1v
        hc = hc * (1.0 / (1.0 + jnp.exp(-hc)))
        oc = jnp.dot(hc.astype(jnp.bfloat16), w2m,
                     preferred_element_type=jnp.float32) + b2v
        out_ref[rows, :] += wcol[i * chunk:(i + 1) * chunk, :] * oc


@jax.jit
def kernel(x, Wr, br, W1, b1, W2, b2):
    B, S, D = x.shape
    x_flat = x.reshape(-1, D)
    wr_pad = jnp.zeros((D, EPAD_), jnp.float32).at[:, :NE_].set(Wr)
    br_pad = jnp.zeros((1, EPAD_), jnp.float32).at[0, :NE_].set(br)

    wfull, aux = pl.pallas_call(
        _router_body,
        grid=(NTOK_ // RTB_,),
        in_specs=[
            pl.BlockSpec((RTB_, DIM_), lambda t: (t, 0)),
            pl.BlockSpec((DIM_, EPAD_), lambda t: (0, 0)),
            pl.BlockSpec((1, EPAD_), lambda t: (0, 0)),
        ],
        out_specs=[
            pl.BlockSpec((RTB_, EPAD_), lambda t: (t, 0)),
            pl.BlockSpec((1, 1), lambda t: (0, 0)),
        ],
        out_shape=[
            jax.ShapeDtypeStruct((NTOK_, EPAD_), jnp.float32),
            jax.ShapeDtypeStruct((1, 1), jnp.float32),
        ],
        scratch_shapes=[pltpu.VMEM((1, EPAD_), jnp.float32)],
        compiler_params=pltpu.CompilerParams(
            dimension_semantics=("arbitrary",)),
    )(x_flat, wr_pad, br_pad)

    out = pl.pallas_call(
        _expert_body,
        grid=(NE_,),
        in_specs=[
            pl.BlockSpec((NTOK_, DIM_), lambda e: (0, 0)),
            pl.BlockSpec((NTOK_, EPAD_), lambda e: (0, 0)),
            pl.BlockSpec((1, DIM_, HID_), lambda e: (e, 0, 0)),
            pl.BlockSpec((1, 1, HID_), lambda e: (e, 0, 0)),
            pl.BlockSpec((1, HID_, DIM_), lambda e: (e, 0, 0)),
            pl.BlockSpec((1, 1, DIM_), lambda e: (e, 0, 0)),
        ],
        out_specs=pl.BlockSpec((NTOK_, DIM_), lambda e: (0, 0)),
        out_shape=jax.ShapeDtypeStruct((NTOK_, DIM_), jnp.float32),
        compiler_params=pltpu.CompilerParams(
            dimension_semantics=("arbitrary",)),
    )(x_flat.astype(jnp.bfloat16), wfull,
      W1.astype(jnp.bfloat16), b1.reshape(NE_, 1, HID_),
      W2.astype(jnp.bfloat16), b2.reshape(NE_, 1, DIM_))

    return out.reshape(B, S, D), aux.reshape(())
